# baseline (speedup 1.0000x reference)
import jax
import jax.numpy as jnp
from jax.experimental import pallas as pl
from jax.experimental.pallas import tpu as pltpu

_ROWS = 2 * 8192
_COLS = 4096
_NCHUNKS = 8


def _dma_body(i_ref, o_ref, sems):
    for c in range(_NCHUNKS):
        pltpu.make_async_copy(i_ref.at[c], o_ref.at[c], sems.at[c]).start()
    for c in range(_NCHUNKS):
        pltpu.make_async_copy(i_ref.at[c], o_ref.at[c], sems.at[c]).wait()


def kernel(x, bit, alpha):
    del bit, alpha
    x3 = x.reshape(_NCHUNKS, _ROWS // _NCHUNKS, _COLS)
    out = pl.pallas_call(
        _dma_body,
        in_specs=[pl.BlockSpec(memory_space=pl.ANY)],
        out_specs=pl.BlockSpec(memory_space=pl.ANY),
        out_shape=jax.ShapeDtypeStruct(x3.shape, x.dtype),
        scratch_shapes=[pltpu.SemaphoreType.DMA((_NCHUNKS,))],
    )(x3)
    return out.reshape(x.shape)


# speedup vs baseline: 12.4507x; 12.4507x over previous
import functools

import jax
import jax.numpy as jnp
from jax import lax
from jax.experimental import pallas as pl
from jax.experimental.pallas import tpu as pltpu
from jax.experimental.pallas import tpu_sc as plsc

_N = 2 * 8192 * 4096
_NW = 32
_PER_W = _N // _NW
_CHUNK = 16384
_G = _PER_W // _CHUNK
_NBUF = 4

_mesh = plsc.VectorSubcoreMesh(core_axis_name="c", subcore_axis_name="s")


@functools.partial(
    pl.kernel,
    mesh=_mesh,
    out_type=jax.ShapeDtypeStruct((_N,), jnp.float32),
    scratch_types=[
        pltpu.VMEM((_NBUF, _CHUNK), jnp.float32),
        pltpu.SemaphoreType.DMA((_NBUF,)),
        pltpu.SemaphoreType.DMA((_NBUF,)),
    ],
)
def _sc_copy(in_hbm, out_hbm, bufs, lsem, ssem):
    wid = lax.axis_index("s") * 2 + lax.axis_index("c")
    base = wid * _PER_W

    for b in range(_NBUF):
        pltpu.make_async_copy(
            in_hbm.at[pl.ds(base + b * _CHUNK, _CHUNK)], bufs.at[b], lsem.at[b]
        ).start()

    def outer(g0, carry):
        for b in range(_NBUF):
            g = g0 * _NBUF + b
            off = base + g * _CHUNK
            pltpu.make_async_copy(
                in_hbm.at[pl.ds(off, _CHUNK)], bufs.at[b], lsem.at[b]
            ).wait()
            pltpu.make_async_copy(
                bufs.at[b], out_hbm.at[pl.ds(off, _CHUNK)], ssem.at[b]
            ).start()

            g2 = g + _NBUF

            @pl.when(g2 < _G)
            def _():
                pltpu.make_async_copy(
                    bufs.at[b], out_hbm.at[pl.ds(off, _CHUNK)], ssem.at[b]
                ).wait()
                pltpu.make_async_copy(
                    in_hbm.at[pl.ds(base + g2 * _CHUNK, _CHUNK)],
                    bufs.at[b],
                    lsem.at[b],
                ).start()

        return carry

    lax.fori_loop(0, _G // _NBUF, outer, 0)

    for b in range(_NBUF):
        off = base + (_G - _NBUF + b) * _CHUNK
        pltpu.make_async_copy(
            bufs.at[b], out_hbm.at[pl.ds(off, _CHUNK)], ssem.at[b]
        ).wait()


def kernel(x, bit, alpha):
    del bit, alpha
    out = _sc_copy(x.reshape(_N))
    return out.reshape(x.shape)


# speedup vs baseline: 44.5760x; 3.5802x over previous
import jax
import jax.numpy as jnp
from jax.experimental import pallas as pl

_ROWS = 2 * 8192
_COLS = 4096
_BLOCK_ROWS = 128


def _copy_body(i_ref, o_ref):
    o_ref[...] = i_ref[...]


def kernel(x, bit, alpha):
    del bit, alpha
    x2 = x.reshape(_ROWS, _COLS)
    out = pl.pallas_call(
        _copy_body,
        grid=(_ROWS // _BLOCK_ROWS,),
        in_specs=[pl.BlockSpec((_BLOCK_ROWS, _COLS), lambda i: (i, 0))],
        out_specs=pl.BlockSpec((_BLOCK_ROWS, _COLS), lambda i: (i, 0)),
        out_shape=jax.ShapeDtypeStruct((_ROWS, _COLS), x.dtype),
    )(x2)
    return out.reshape(x.shape)


# speedup vs baseline: 48.9967x; 1.0992x over previous
import jax
import jax.numpy as jnp
from jax.experimental import pallas as pl

_ROWS = 2 * 8192
_COLS = 4096
_BLOCK_ROWS = 512


def _copy_body(i_ref, o_ref):
    o_ref[...] = i_ref[...]


def kernel(x, bit, alpha):
    del bit, alpha
    x2 = x.reshape(_ROWS, _COLS)
    out = pl.pallas_call(
        _copy_body,
        grid=(_ROWS // _BLOCK_ROWS,),
        in_specs=[pl.BlockSpec((_BLOCK_ROWS, _COLS), lambda i: (i, 0))],
        out_specs=pl.BlockSpec((_BLOCK_ROWS, _COLS), lambda i: (i, 0)),
        out_shape=jax.ShapeDtypeStruct((_ROWS, _COLS), x.dtype),
    )(x2)
    return out.reshape(x.shape)
